# baseline (speedup 1.0000x reference)
import functools

import jax
import jax.numpy as jnp
from jax import lax
from jax.experimental import pallas as pl
from jax.experimental.pallas import tpu as pltpu
from jax.experimental.pallas import tpu_sc as plsc

N_NODES = 10000
N_EDGES = 320000
D = 128

NC = 2
NS = 16
NW = NC * NS
EDGES_PER_W = N_EDGES // NW
CHUNK = 80
NCHUNK = EDGES_PER_W // CHUNK
NBUF = 4
DRING = 8
SRING = 8
UNROLL = 8
NGROUP = NCHUNK // UNROLL
ACC_ROWS = 10240
ROWS_PER_TILE = ACC_ROWS // NS


def _sc_segment_sum(x, src_r, dst_r):
    mesh = plsc.VectorSubcoreMesh(
        core_axis_name="c", subcore_axis_name="s", num_cores=NC, num_subcores=NS
    )

    @functools.partial(
        pl.kernel,
        out_type=jax.ShapeDtypeStruct((NC, ACC_ROWS, D), jnp.float32),
        mesh=mesh,
        scratch_types=[
            pltpu.VMEM((SRING * CHUNK,), jnp.int32),
            pltpu.VMEM((DRING, CHUNK), jnp.int32),
            pltpu.VMEM((CHUNK, D), jnp.float32),
            pltpu.VMEM((CHUNK, D), jnp.float32),
            pltpu.VMEM((CHUNK, D), jnp.float32),
            pltpu.VMEM((CHUNK, D), jnp.float32),
            pltpu.VMEM_SHARED((ACC_ROWS, D), jnp.float32),
            pltpu.SemaphoreType.DMA((NBUF,)),
            pltpu.SemaphoreType.DMA((NBUF,)),
            pltpu.SemaphoreType.DMA((DRING,)),
            pltpu.SemaphoreType.DMA((SRING,)),
        ],
    )
    def k(x_hbm, src_hbm, dst_hbm, out_hbm,
          src_v, dst_ring, rows0, rows1, rows2, rows3, acc,
          gsems, scsems, dsems, ssems):
        bufs = (rows0, rows1, rows2, rows3)
        c = lax.axis_index("c")
        s = lax.axis_index("s")
        wid = s * NC + c
        ebase = pl.multiple_of(wid * EDGES_PER_W, 8)

        def _off(j):
            o = j * CHUNK
            if not isinstance(o, int):
                o = pl.multiple_of(o, 8)
            return pl.multiple_of(ebase + o, 8)

        def start_dst(j, u):
            pltpu.async_copy(
                dst_hbm.at[pl.ds(_off(j), CHUNK)],
                dst_ring.at[u],
                dsems.at[u],
            )

        def wait_dst(u):
            pltpu.make_async_copy(
                dst_hbm.at[pl.ds(0, CHUNK)], dst_ring.at[u], dsems.at[u]
            ).wait()

        def start_src(j, sl):
            pltpu.async_copy(
                src_hbm.at[pl.ds(_off(j), CHUNK)],
                src_v.at[pl.ds(sl * CHUNK, CHUNK)],
                ssems.at[sl],
            )

        def wait_src(sl):
            pltpu.make_async_copy(
                src_hbm.at[pl.ds(0, CHUNK)],
                src_v.at[pl.ds(sl * CHUNK, CHUNK)],
                ssems.at[sl],
            ).wait()

        def start_gather(j, bb, sl):
            pltpu.async_copy(
                x_hbm.at[src_v.at[pl.ds(sl * CHUNK, CHUNK)]],
                bufs[bb],
                gsems.at[bb],
            )

        def wait_gather(bb):
            pltpu.make_async_copy(
                x_hbm.at[src_v.at[pl.ds(0, CHUNK)]], bufs[bb], gsems.at[bb]
            ).wait()

        def start_scatter(bb, u):
            pltpu.async_copy(bufs[bb], acc.at[dst_ring.at[u]], scsems.at[bb], add=True)

        def wait_scatter(bb):
            pltpu.make_async_copy(
                bufs[bb], acc.at[dst_ring.at[0]], scsems.at[bb]
            ).wait()

        for u in range(6):
            start_dst(u, u)
            start_src(u, u)

        zeros16 = jnp.zeros((16,), jnp.float32)

        def zero_row(r, _):
            for kk in range(D // 16):
                rows0[r, pl.ds(kk * 16, 16)] = zeros16
            return _

        lax.fori_loop(0, CHUNK, zero_row, None)
        base = s * ROWS_PER_TILE
        for blk in range(ROWS_PER_TILE // CHUNK):
            pltpu.sync_copy(rows0, acc.at[pl.ds(base + blk * CHUNK, CHUNK)])
        plsc.subcore_barrier()

        for bb in range(2):
            wait_src(bb)
            start_gather(bb, bb, bb)

        def chunk_step(j, u, jj):
            bb = u % NBUF

            def maybe(pred, fn):
                if jj is not None:
                    if pred(jj):
                        fn()
                else:
                    pl.when(pred(j))(fn)

            wait_gather(bb)
            wait_dst(u % DRING)
            start_scatter(bb, u % DRING)
            maybe(lambda t: t >= 2, lambda: wait_scatter((bb + 2) % NBUF))
            maybe(lambda t: t + 6 < NCHUNK,
                  lambda: start_dst(j + 6, (u + 6) % DRING))
            maybe(lambda t: t + 6 < NCHUNK,
                  lambda: start_src(j + 6, (u + 6) % SRING))

            def _next_gather():
                wait_src((u + 2) % SRING)
                start_gather(j + 2, (bb + 2) % NBUF, (u + 2) % SRING)

            maybe(lambda t: t + 2 < NCHUNK, _next_gather)

        def group(g, _):
            for u in range(UNROLL):
                chunk_step(g * UNROLL + u, u, None)
            return _

        lax.fori_loop(0, NGROUP, group, None)
        for j in range(NGROUP * UNROLL, NCHUNK):
            chunk_step(j, j % UNROLL, j)
        wait_scatter((NCHUNK - 2) % NBUF)
        wait_scatter((NCHUNK - 1) % NBUF)
        plsc.subcore_barrier()

        pltpu.sync_copy(
            acc.at[pl.ds(base, ROWS_PER_TILE)],
            out_hbm.at[c, pl.ds(base, ROWS_PER_TILE)],
        )

    return k(x, src_r, dst_r)


def _tc_linear(partials, W, b):
    BLK = 1000

    def body(p0_ref, p1_ref, w_ref, b_ref, out_ref):
        h = p0_ref[0] + p1_ref[0]
        out_ref[...] = (
            jax.lax.dot_general(
                h, w_ref[...], (((1,), (1,)), ((), ())),
                preferred_element_type=jnp.float32,
            )
            + b_ref[...]
        )

    return pl.pallas_call(
        body,
        out_shape=jax.ShapeDtypeStruct((N_NODES, D), jnp.float32),
        grid=(N_NODES // BLK,),
        in_specs=[
            pl.BlockSpec((1, BLK, D), lambda i: (0, i, 0)),
            pl.BlockSpec((1, BLK, D), lambda i: (1, i, 0)),
            pl.BlockSpec((D, D), lambda i: (0, 0)),
            pl.BlockSpec((1, D), lambda i: (0, 0)),
        ],
        out_specs=pl.BlockSpec((BLK, D), lambda i: (i, 0)),
    )(partials, partials, W, b)


@jax.jit
def kernel(x, edge_index, W, b):
    src = edge_index[0].astype(jnp.int32)
    dst = edge_index[1].astype(jnp.int32)
    partials = _sc_segment_sum(x, src, dst)
    return _tc_linear(partials, W, b.reshape(1, D))


# speedup vs baseline: 1.1358x; 1.1358x over previous
import functools

import jax
import jax.numpy as jnp
from jax import lax
from jax.experimental import pallas as pl
from jax.experimental.pallas import tpu as pltpu
from jax.experimental.pallas import tpu_sc as plsc

N_NODES = 10000
N_EDGES = 320000
D = 128

NC = 2
NS = 16
NW = NC * NS
EDGES_PER_W = N_EDGES // NW
CHUNK = 80
NCHUNK = EDGES_PER_W // CHUNK
NBUF = 3
DRING = 6
UNROLL = 6
NGROUP = NCHUNK // UNROLL
ACC_ROWS = 10240
ROWS_PER_TILE = ACC_ROWS // NS


def _sc_segment_sum(x, src_r, dst_r):
    mesh = plsc.VectorSubcoreMesh(
        core_axis_name="c", subcore_axis_name="s", num_cores=NC, num_subcores=NS
    )

    @functools.partial(
        pl.kernel,
        out_type=jax.ShapeDtypeStruct((NC, ACC_ROWS, D), jnp.float32),
        mesh=mesh,
        scratch_types=[
            pltpu.VMEM((EDGES_PER_W,), jnp.int32),
            pltpu.VMEM((DRING, CHUNK), jnp.int32),
            pltpu.VMEM((CHUNK, D), jnp.float32),
            pltpu.VMEM((CHUNK, D), jnp.float32),
            pltpu.VMEM((CHUNK, D), jnp.float32),
            pltpu.VMEM_SHARED((ACC_ROWS, D), jnp.float32),
            pltpu.SemaphoreType.DMA((NBUF,)),
            pltpu.SemaphoreType.DMA((NBUF,)),
            pltpu.SemaphoreType.DMA((DRING,)),
            pltpu.SemaphoreType.DMA,
        ],
    )
    def k(x_hbm, src_hbm, dst_hbm, out_hbm,
          src_v, dst_ring, rows0, rows1, rows2, acc, gsems, scsems, dsems, isem):
        bufs = (rows0, rows1, rows2)
        c = lax.axis_index("c")
        s = lax.axis_index("s")
        wid = s * NC + c

        ebase = pl.multiple_of(wid * EDGES_PER_W, 8)
        pltpu.async_copy(src_hbm.at[pl.ds(ebase, EDGES_PER_W)], src_v, isem)

        def _off(j):
            o = j * CHUNK
            return o if isinstance(o, int) else pl.multiple_of(o, 8)

        def start_dst(j, u):
            pltpu.async_copy(
                dst_hbm.at[pl.ds(pl.multiple_of(ebase + j * CHUNK, 8), CHUNK)],
                dst_ring.at[u],
                dsems.at[u],
            )

        def wait_dst(u):
            pltpu.make_async_copy(
                dst_hbm.at[pl.ds(0, CHUNK)], dst_ring.at[u], dsems.at[u]
            ).wait()

        for u in range(DRING - 1):
            start_dst(u, u)

        zeros16 = jnp.zeros((16,), jnp.float32)

        def zero_row(r, _):
            for kk in range(D // 16):
                rows0[r, pl.ds(kk * 16, 16)] = zeros16
            return _

        lax.fori_loop(0, CHUNK, zero_row, None)
        base = s * ROWS_PER_TILE
        nz = ROWS_PER_TILE // CHUNK
        for blk in range(nz):
            pltpu.async_copy(
                rows0, acc.at[pl.ds(base + blk * CHUNK, CHUNK)], gsems.at[blk % 2]
            )
        for blk in range(nz):
            pltpu.make_async_copy(
                rows0, acc.at[pl.ds(base + blk * CHUNK, CHUNK)], gsems.at[blk % 2]
            ).wait()
        pltpu.make_async_copy(src_hbm.at[pl.ds(0, EDGES_PER_W)], src_v, isem).wait()
        plsc.subcore_barrier()

        def start_gather(j, bb):
            pltpu.async_copy(
                x_hbm.at[src_v.at[pl.ds(_off(j), CHUNK)]],
                bufs[bb],
                gsems.at[bb],
            )

        def wait_gather(bb):
            pltpu.make_async_copy(
                x_hbm.at[src_v.at[pl.ds(0, CHUNK)]], bufs[bb], gsems.at[bb]
            ).wait()

        def start_scatter(bb, u):
            pltpu.async_copy(bufs[bb], acc.at[dst_ring.at[u]], scsems.at[bb], add=True)

        def wait_scatter(bb):
            pltpu.make_async_copy(
                bufs[bb], acc.at[dst_ring.at[0]], scsems.at[bb]
            ).wait()

        start_gather(0, 0)
        start_gather(1, 1)

        def chunk_step(j, u, jj):
            bb = u % NBUF

            def maybe(pred, fn):
                if jj is not None:
                    if pred(jj):
                        fn()
                else:
                    pl.when(pred(j))(fn)

            wait_gather(bb)
            wait_dst(u)
            start_scatter(bb, u)
            maybe(lambda t: t >= 1, lambda: wait_scatter((bb + 2) % NBUF))
            maybe(lambda t: t + DRING - 1 < NCHUNK,
                  lambda: start_dst(j + DRING - 1, (u + DRING - 1) % DRING))
            maybe(lambda t: t + 2 < NCHUNK,
                  lambda: start_gather(j + 2, (bb + 2) % NBUF))

        def group(g, _):
            for u in range(UNROLL):
                chunk_step(g * UNROLL + u, u, None)
            return _

        lax.fori_loop(0, NGROUP, group, None)
        for j in range(NGROUP * UNROLL, NCHUNK):
            chunk_step(j, j % DRING, j)
        wait_scatter((NCHUNK - 1) % NBUF)
        plsc.subcore_barrier()

        pltpu.sync_copy(
            acc.at[pl.ds(base, ROWS_PER_TILE)],
            out_hbm.at[c, pl.ds(base, ROWS_PER_TILE)],
        )

    return k(x, src_r, dst_r)


def _tc_linear(partials, W, b):
    BLK = 2000

    def body(p0_ref, p1_ref, w_ref, b_ref, out_ref):
        h = p0_ref[0] + p1_ref[0]
        out_ref[...] = (
            jax.lax.dot_general(
                h, w_ref[...], (((1,), (1,)), ((), ())),
                preferred_element_type=jnp.float32,
            )
            + b_ref[...]
        )

    return pl.pallas_call(
        body,
        out_shape=jax.ShapeDtypeStruct((N_NODES, D), jnp.float32),
        grid=(N_NODES // BLK,),
        in_specs=[
            pl.BlockSpec((1, BLK, D), lambda i: (0, i, 0)),
            pl.BlockSpec((1, BLK, D), lambda i: (1, i, 0)),
            pl.BlockSpec((D, D), lambda i: (0, 0)),
            pl.BlockSpec((1, D), lambda i: (0, 0)),
        ],
        out_specs=pl.BlockSpec((BLK, D), lambda i: (i, 0)),
    )(partials, partials, W, b)


@jax.jit
def kernel(x, edge_index, W, b):
    src = edge_index[0].astype(jnp.int32)
    dst = edge_index[1].astype(jnp.int32)
    partials = _sc_segment_sum(x, src, dst)
    return _tc_linear(partials, W, b.reshape(1, D))
